# baseline (speedup 1.0000x reference)
import functools

import jax
import jax.numpy as jnp
from jax import lax
from jax.experimental import pallas as pl
from jax.experimental.pallas import tpu as pltpu
from jax.experimental.pallas import tpu_sc as plsc


def _make_gather(B, V, D, n_workers, chunk, nbuf, fd):
    per_w = B // n_workers
    n_chunks = per_w // chunk
    mesh = plsc.VectorSubcoreMesh(core_axis_name="c", subcore_axis_name="s")
    nc = mesh.num_cores

    @functools.partial(
        pl.kernel,
        mesh=mesh,
        out_type=jax.ShapeDtypeStruct((n_workers * n_chunks, chunk, D), jnp.float32),
        scratch_types=[
            pltpu.VMEM((n_chunks, chunk), jnp.int32),
            pltpu.VMEM((nbuf, chunk, D), jnp.float32),
            pltpu.SemaphoreType.DMA((nbuf,)),
            pltpu.SemaphoreType.DMA((nbuf,)),
        ],
    )
    def gather_kernel(idx_hbm, table_hbm, out_hbm, idx_v, buf, gsem, wsem):
        wid = lax.axis_index("s") * nc + lax.axis_index("c")
        row0 = wid * n_chunks
        pltpu.sync_copy(idx_hbm.at[wid], idx_v)

        def fire(j):
            b = lax.rem(j, nbuf)
            pltpu.async_copy(table_hbm.at[idx_v.at[j]], buf.at[b], gsem.at[b])

        for t in range(fd):
            fire(t)

        @pl.loop(0, n_chunks)
        def _(j):
            b = lax.rem(j, nbuf)
            pltpu.make_async_copy(table_hbm.at[idx_v.at[j]], buf.at[b], gsem.at[b]).wait()
            pltpu.async_copy(buf.at[b], out_hbm.at[row0 + j], wsem.at[b])

            @pl.when(j + fd < n_chunks)
            def _():
                bn = lax.rem(j + fd, nbuf)

                @pl.when(j >= nbuf - fd)
                def _():
                    pltpu.make_async_copy(
                        buf.at[bn], out_hbm.at[row0 + j + fd - nbuf], wsem.at[bn]
                    ).wait()

                fire(j + fd)

        for t in range(nbuf):
            j = n_chunks - nbuf + t
            b = j % nbuf
            pltpu.make_async_copy(buf.at[b], out_hbm.at[row0 + j], wsem.at[b]).wait()

    return gather_kernel


def kernel(x, weight):
    Bx, S = x.shape
    V, D = weight.shape
    B = Bx * S
    chunk = 64
    n_workers = 32
    nbuf = 12
    fd = 6
    n_chunks = B // (n_workers * chunk)
    idx = x.reshape(n_workers, n_chunks, chunk).astype(jnp.int32)
    out = _make_gather(B, V, D, n_workers, chunk, nbuf, fd)(idx, weight)
    return out.reshape(Bx, S, D)


# speedup vs baseline: 1.0050x; 1.0050x over previous
import functools

import jax
import jax.numpy as jnp
from jax import lax
from jax.experimental import pallas as pl
from jax.experimental.pallas import tpu as pltpu
from jax.experimental.pallas import tpu_sc as plsc


def _make_gather(B, V, D, n_workers, chunk, nbuf, fd):
    per_w = B // n_workers
    n_chunks = per_w // chunk
    mesh = plsc.VectorSubcoreMesh(core_axis_name="c", subcore_axis_name="s")
    nc = mesh.num_cores

    @functools.partial(
        pl.kernel,
        mesh=mesh,
        out_type=jax.ShapeDtypeStruct((n_workers * n_chunks, chunk, D), jnp.float32),
        scratch_types=[
            pltpu.VMEM((n_chunks, chunk), jnp.int32),
            pltpu.VMEM((nbuf, chunk, D), jnp.float32),
            pltpu.SemaphoreType.DMA((nbuf,)),
            pltpu.SemaphoreType.DMA((nbuf,)),
        ],
    )
    def gather_kernel(idx_hbm, table_hbm, out_hbm, idx_v, buf, gsem, wsem):
        wid = lax.axis_index("s") * nc + lax.axis_index("c")
        row0 = wid * n_chunks
        pltpu.sync_copy(idx_hbm.at[wid], idx_v)

        def fire(j):
            b = lax.rem(j, nbuf)
            pltpu.async_copy(table_hbm.at[idx_v.at[j]], buf.at[b], gsem.at[b])

        for t in range(fd):
            fire(t)

        @pl.loop(0, n_chunks)
        def _(j):
            b = lax.rem(j, nbuf)
            pltpu.make_async_copy(table_hbm.at[idx_v.at[j]], buf.at[b], gsem.at[b]).wait()
            pltpu.async_copy(buf.at[b], out_hbm.at[row0 + j], wsem.at[b])

            @pl.when(j + fd < n_chunks)
            def _():
                bn = lax.rem(j + fd, nbuf)

                @pl.when(j >= nbuf - fd)
                def _():
                    pltpu.make_async_copy(
                        buf.at[bn], out_hbm.at[row0 + j + fd - nbuf], wsem.at[bn]
                    ).wait()

                fire(j + fd)

        for t in range(nbuf):
            j = n_chunks - nbuf + t
            b = j % nbuf
            pltpu.make_async_copy(buf.at[b], out_hbm.at[row0 + j], wsem.at[b]).wait()

    return gather_kernel


def kernel(x, weight):
    Bx, S = x.shape
    V, D = weight.shape
    B = Bx * S
    chunk = 128
    n_workers = 32
    nbuf = 7
    fd = 5
    n_chunks = B // (n_workers * chunk)
    idx = x.reshape(n_workers, n_chunks, chunk).astype(jnp.int32)
    out = _make_gather(B, V, D, n_workers, chunk, nbuf, fd)(idx, weight)
    return out.reshape(Bx, S, D)


# speedup vs baseline: 1.0104x; 1.0054x over previous
import functools

import jax
import jax.numpy as jnp
from jax import lax
from jax.experimental import pallas as pl
from jax.experimental.pallas import tpu as pltpu
from jax.experimental.pallas import tpu_sc as plsc


def _make_gather(B, V, D, n_workers, chunk, nbuf, fd):
    per_w = B // n_workers
    n_chunks = per_w // chunk
    mesh = plsc.VectorSubcoreMesh(core_axis_name="c", subcore_axis_name="s")
    nc = mesh.num_cores

    @functools.partial(
        pl.kernel,
        mesh=mesh,
        out_type=jax.ShapeDtypeStruct((n_workers * n_chunks, chunk, D), jnp.float32),
        scratch_types=[
            pltpu.VMEM((n_chunks, chunk), jnp.int32),
            pltpu.VMEM((nbuf, chunk, D), jnp.float32),
            pltpu.SemaphoreType.DMA((nbuf,)),
            pltpu.SemaphoreType.DMA((nbuf,)),
        ],
    )
    def gather_kernel(idx_hbm, table_hbm, out_hbm, idx_v, buf, gsem, wsem):
        wid = lax.axis_index("s") * nc + lax.axis_index("c")
        row0 = wid * n_chunks
        pltpu.sync_copy(idx_hbm.at[wid], idx_v)

        def fire(j):
            b = lax.rem(j, nbuf)
            pltpu.async_copy(table_hbm.at[idx_v.at[j]], buf.at[b], gsem.at[b])

        for t in range(fd):
            fire(t)

        @pl.loop(0, n_chunks)
        def _(j):
            b = lax.rem(j, nbuf)
            pltpu.make_async_copy(table_hbm.at[idx_v.at[j]], buf.at[b], gsem.at[b]).wait()
            pltpu.async_copy(buf.at[b], out_hbm.at[row0 + j], wsem.at[b])

            @pl.when(j + fd < n_chunks)
            def _():
                bn = lax.rem(j + fd, nbuf)

                @pl.when(j >= nbuf - fd)
                def _():
                    pltpu.make_async_copy(
                        buf.at[bn], out_hbm.at[row0 + j + fd - nbuf], wsem.at[bn]
                    ).wait()

                fire(j + fd)

        for t in range(nbuf):
            j = n_chunks - nbuf + t
            b = j % nbuf
            pltpu.make_async_copy(buf.at[b], out_hbm.at[row0 + j], wsem.at[b]).wait()

    return gather_kernel


def kernel(x, weight):
    Bx, S = x.shape
    V, D = weight.shape
    B = Bx * S
    chunk = 128
    n_workers = 32
    nbuf = 7
    fd = 4
    n_chunks = B // (n_workers * chunk)
    idx = x.reshape(n_workers, n_chunks, chunk).astype(jnp.int32)
    out = _make_gather(B, V, D, n_workers, chunk, nbuf, fd)(idx, weight)
    return out.reshape(Bx, S, D)
